# Initial kernel scaffold; baseline (speedup 1.0000x reference)
import jax
import jax.numpy as jnp
from jax.experimental import pallas as pl


def kernel(x, pe_table):
    raise NotImplementedError("write your pallas kernel here")


# speedup vs baseline: 1.6702x; 1.6702x over previous
import jax
import jax.numpy as jnp
from jax.experimental import pallas as pl

S_BLK = 256


def _add_kernel(x_ref, pe_ref, o_ref):
    o_ref[...] = x_ref[...] + pe_ref[...]


def kernel(x, pe_table):
    batch, seq_len, embed_dim = x.shape
    n_s = seq_len // S_BLK
    return pl.pallas_call(
        _add_kernel,
        grid=(n_s, batch),
        in_specs=[
            pl.BlockSpec((1, S_BLK, embed_dim), lambda s, b: (b, s, 0)),
            pl.BlockSpec((S_BLK, embed_dim), lambda s, b: (s, 0)),
        ],
        out_specs=pl.BlockSpec((1, S_BLK, embed_dim), lambda s, b: (b, s, 0)),
        out_shape=jax.ShapeDtypeStruct(x.shape, x.dtype),
    )(x, pe_table)


# speedup vs baseline: 1.8498x; 1.1076x over previous
import jax
import jax.numpy as jnp
from jax.experimental import pallas as pl

S_BLK = 512


def _add_kernel(x_ref, pe_ref, o_ref):
    o_ref[...] = x_ref[...] + pe_ref[...]


def kernel(x, pe_table):
    batch, seq_len, embed_dim = x.shape
    n_s = seq_len // S_BLK
    return pl.pallas_call(
        _add_kernel,
        grid=(n_s, batch),
        in_specs=[
            pl.BlockSpec((1, S_BLK, embed_dim), lambda s, b: (b, s, 0)),
            pl.BlockSpec((S_BLK, embed_dim), lambda s, b: (s, 0)),
        ],
        out_specs=pl.BlockSpec((1, S_BLK, embed_dim), lambda s, b: (b, s, 0)),
        out_shape=jax.ShapeDtypeStruct(x.shape, x.dtype),
    )(x, pe_table)


# speedup vs baseline: 1.9705x; 1.0653x over previous
import jax
import jax.numpy as jnp
from jax.experimental import pallas as pl

S_BLK = 1024


def _add_kernel(x_ref, pe_ref, o_ref):
    o_ref[...] = x_ref[...] + pe_ref[...]


def kernel(x, pe_table):
    batch, seq_len, embed_dim = x.shape
    n_s = seq_len // S_BLK
    return pl.pallas_call(
        _add_kernel,
        grid=(n_s, batch),
        in_specs=[
            pl.BlockSpec((1, S_BLK, embed_dim), lambda s, b: (b, s, 0)),
            pl.BlockSpec((S_BLK, embed_dim), lambda s, b: (s, 0)),
        ],
        out_specs=pl.BlockSpec((1, S_BLK, embed_dim), lambda s, b: (b, s, 0)),
        out_shape=jax.ShapeDtypeStruct(x.shape, x.dtype),
    )(x, pe_table)
